# baseline (speedup 1.0000x reference)
import functools

import jax
import jax.numpy as jnp
from jax import lax
from jax.experimental import pallas as pl
from jax.experimental.pallas import tpu as pltpu
from jax.experimental.pallas import tpu_sc as plsc

NC = 2
NS = 16
NW = NC * NS
CHUNK = 128
NBUF = 4
EMB = 64
LANES = 16


@functools.cache
def _build(n_seq: int, n_tok: int):
    sblocks = n_seq // CHUNK
    sb_per_w = sblocks // NW
    nbody = n_tok // NBUF - 1
    rem = n_tok - (nbody + 1) * NBUF
    assert sb_per_w * NW == sblocks and nbody >= 1 and 0 <= rem < NBUF

    mesh = plsc.VectorSubcoreMesh(core_axis_name="c", subcore_axis_name="s")

    @functools.partial(
        pl.kernel,
        mesh=mesh,
        out_type=jax.ShapeDtypeStruct(
            (n_tok, EMB // 8, sblocks, 8, CHUNK), jnp.float32),
        scratch_types=(
            [pltpu.VMEM((CHUNK, n_tok), jnp.int32),
             pltpu.VMEM((n_tok, CHUNK), jnp.int32),
             pltpu.VMEM((n_tok, CHUNK), jnp.int32)]
            + [pltpu.VMEM((CHUNK, 2 * EMB), jnp.float32) for _ in range(NBUF)]
            + [pltpu.VMEM((EMB // 8, 8, CHUNK + 1), jnp.float32)
               for _ in range(NBUF)]
            + [pltpu.SemaphoreType.DMA for _ in range(2 * NBUF)]
        ),
        compiler_params=pltpu.CompilerParams(use_tc_tiling_on_sc=False,
                                             needs_layout_passes=False),
    )
    def emb(tok_hbm, w2_hbm, out_hbm, tkb, idxs, offs, *rest):
        rows = rest[:NBUF]
        tes = rest[NBUF:2 * NBUF]
        gsem = rest[2 * NBUF:3 * NBUF]
        wsem = rest[3 * NBUF:]
        wid = lax.axis_index("s") * NC + lax.axis_index("c")
        lane = lax.broadcasted_iota(jnp.int32, (LANES,), 0)

        ebv = [(lane + evb * LANES) // 8 for evb in range(EMB // LANES)]
        eiv = [(lane + evb * LANES) % 8 for evb in range(EMB // LANES)]

        def transpose(t, m):
            def tbody(q, carry):
                ovq = offs[t, pl.ds(q * LANES, LANES)]
                for u in range(LANES):
                    si = q * LANES + u
                    off = jnp.sum(jnp.where(lane == u, ovq, 0))
                    vs = [rows[m][si, pl.ds(off + evb * LANES, LANES)]
                          for evb in range(EMB // LANES)]
                    siv = jnp.full((LANES,), si, jnp.int32)
                    for evb in range(EMB // LANES):
                        plsc.store_scatter(tes[m], [ebv[evb], eiv[evb], siv],
                                           vs[evb])
                return carry

            lax.fori_loop(0, CHUNK // LANES, tbody, 0)

        def phase(sbi, carry):
            sb = wid * sb_per_w + sbi
            pltpu.sync_copy(tok_hbm.at[pl.ds(sb * CHUNK, CHUNK)], tkb)

            def ib(t, c):
                tv = jnp.full((LANES,), t, jnp.int32)
                vs = [plsc.load_gather(tkb, [lane + sv * LANES, tv])
                      for sv in range(CHUNK // LANES)]
                for sv in range(CHUNK // LANES):
                    idxs[t, pl.ds(sv * LANES, LANES)] = vs[sv] >> 1
                    offs[t, pl.ds(sv * LANES, LANES)] = (vs[sv] & 1) << 6
                return c

            lax.fori_loop(0, n_tok, ib, 0)

            def fire(t, m):
                pltpu.async_copy(w2_hbm.at[idxs.at[t]], rows[m], gsem[m])

            def drain_gather(m):
                pltpu.make_async_copy(w2_hbm.at[pl.ds(0, CHUNK)], rows[m],
                                      gsem[m]).wait()

            def start_write(t, m):
                for eb in range(EMB // 8):
                    pltpu.async_copy(tes[m].at[eb, :, pl.ds(0, CHUNK)],
                                     out_hbm.at[t, eb, sb], wsem[m])

            def drain_write(t, m):
                for eb in range(EMB // 8):
                    pltpu.make_async_copy(tes[m].at[eb, :, pl.ds(0, CHUNK)],
                                          out_hbm.at[t, eb, sb],
                                          wsem[m]).wait()

            for m in range(NBUF):
                fire(m, m)

            def body(j, c):
                t0 = j * NBUF
                for m in range(NBUF):
                    drain_gather(m)

                    @pl.when(j >= 1)
                    def _():
                        drain_write(t0 + m - NBUF, m)

                    transpose(t0 + m, m)
                    start_write(t0 + m, m)
                    fire(t0 + NBUF + m, m)
                return c

            lax.fori_loop(0, nbody, body, 0)

            t0 = nbody * NBUF
            for m in range(NBUF):
                drain_gather(m)
                drain_write(t0 + m - NBUF, m)
                transpose(t0 + m, m)
                start_write(t0 + m, m)
            for m in range(rem):
                drain_write(t0 + m, m)
                fire(t0 + NBUF + m, m)
            for m in range(rem):
                drain_gather(m)
                transpose(t0 + NBUF + m, m)
                start_write(t0 + NBUF + m, m)
            for m in range(rem, NBUF):
                drain_write(t0 + m, m)
            for m in range(rem):
                drain_write(t0 + NBUF + m, m)
            return carry

        lax.fori_loop(0, sb_per_w, phase, 0)

    return emb


def kernel(token_ids, weight):
    s, t = token_ids.shape
    w2 = weight.reshape(weight.shape[0] // 2, 2 * EMB)
    o5 = _build(s, t)(token_ids.astype(jnp.int32), w2)
    return o5.transpose(2, 4, 0, 1, 3).reshape(s, t, EMB)


# speedup vs baseline: 1.0834x; 1.0834x over previous
import functools

import jax
import jax.numpy as jnp
from jax import lax
from jax.experimental import pallas as pl
from jax.experimental.pallas import tpu as pltpu
from jax.experimental.pallas import tpu_sc as plsc

NC = 2
NS = 16
NW = NC * NS
CHUNK = 128
NBUF = 5
EMB = 64
LANES = 16


@functools.cache
def _build(n_seq: int, n_tok: int):
    sblocks = n_seq // CHUNK
    sb_per_w = sblocks // NW
    tpad = (n_tok + 7) // 8
    nout = n_tok // NBUF
    assert sb_per_w * NW == sblocks and nout * NBUF == n_tok and nout >= 2

    mesh = plsc.VectorSubcoreMesh(core_axis_name="c", subcore_axis_name="s")

    @functools.partial(
        pl.kernel,
        mesh=mesh,
        out_type=jax.ShapeDtypeStruct(
            (n_tok, EMB // 8, sblocks, 8, CHUNK), jnp.float32),
        scratch_types=(
            [pltpu.VMEM((tpad, 8, CHUNK), jnp.int32)]
            + [pltpu.VMEM((CHUNK, EMB), jnp.float32) for _ in range(NBUF)]
            + [pltpu.VMEM((EMB // 8, 8, CHUNK + 1), jnp.float32)
               for _ in range(NBUF)]
            + [pltpu.SemaphoreType.DMA for _ in range(2 * NBUF)]
        ),
        compiler_params=pltpu.CompilerParams(use_tc_tiling_on_sc=False,
                                             needs_layout_passes=False),
    )
    def emb(tok_hbm, w_hbm, out_hbm, vtok, *rest):
        rows = rest[:NBUF]
        tes = rest[NBUF:2 * NBUF]
        gsem = rest[2 * NBUF:3 * NBUF]
        wsem = rest[3 * NBUF:]
        wid = lax.axis_index("s") * NC + lax.axis_index("c")
        lane = lax.broadcasted_iota(jnp.int32, (LANES,), 0)

        ebv = [(lane + evb * LANES) // 8 for evb in range(EMB // LANES)]
        eiv = [(lane + evb * LANES) % 8 for evb in range(EMB // LANES)]

        def transpose(m):
            def tbody(q, carry):
                for u in range(8):
                    si = q * 8 + u
                    vs = [rows[m][si, pl.ds(evb * LANES, LANES)]
                          for evb in range(EMB // LANES)]
                    siv = jnp.full((LANES,), si, jnp.int32)
                    for evb in range(EMB // LANES):
                        plsc.store_scatter(tes[m], [ebv[evb], eiv[evb], siv],
                                           vs[evb])
                return carry

            lax.fori_loop(0, CHUNK // 8, tbody, 0)

        def phase(sbi, carry):
            sb = wid * sb_per_w + sbi
            pltpu.sync_copy(tok_hbm.at[:, sb], vtok)

            def fire(t, m):
                pltpu.async_copy(w_hbm.at[vtok.at[t // 8, t % 8]], rows[m],
                                 gsem[m])

            def drain_gather(m):
                pltpu.make_async_copy(w_hbm.at[pl.ds(0, CHUNK)], rows[m],
                                      gsem[m]).wait()

            def start_write(t, m):
                for eb in range(EMB // 8):
                    pltpu.async_copy(tes[m].at[eb, :, pl.ds(0, CHUNK)],
                                     out_hbm.at[t, eb, sb], wsem[m])

            def drain_write(t, m):
                for eb in range(EMB // 8):
                    pltpu.make_async_copy(tes[m].at[eb, :, pl.ds(0, CHUNK)],
                                          out_hbm.at[t, eb, sb],
                                          wsem[m]).wait()

            for m in range(NBUF):
                fire(m, m)

            def body(j, c):
                t0 = j * NBUF
                for m in range(NBUF):
                    drain_gather(m)

                    @pl.when(j >= 1)
                    def _():
                        drain_write(t0 + m - NBUF, m)

                    transpose(m)
                    start_write(t0 + m, m)
                    fire(t0 + NBUF + m, m)
                return c

            lax.fori_loop(0, nout - 1, body, 0)

            t0 = (nout - 1) * NBUF
            for m in range(NBUF):
                drain_gather(m)
                drain_write(t0 + m - NBUF, m)
                transpose(m)
                start_write(t0 + m, m)
            for m in range(NBUF):
                drain_write(t0 + m, m)
            return carry

        lax.fori_loop(0, sb_per_w, phase, 0)

    return emb


def kernel(token_ids, weight):
    s, t = token_ids.shape
    tpad = (t + 7) // 8
    t4 = jnp.pad(token_ids.astype(jnp.int32).T, ((0, tpad * 8 - t), (0, 0)))
    t4 = t4.reshape(tpad, 8, s // CHUNK, CHUNK).transpose(0, 2, 1, 3)
    o5 = _build(s, t)(t4, weight)
    return o5.transpose(2, 4, 0, 1, 3).reshape(s, t, EMB)
